# baseline (speedup 1.0000x reference)
import functools

import jax
import jax.numpy as jnp
from jax import lax
from jax.experimental import pallas as pl
from jax.experimental.pallas import tpu as pltpu
from jax.experimental.pallas import tpu_sc as plsc

_C = 128
_DIMS = (64, 32, 16, 32)


def _make_gather(B):
    info = plsc.get_sparse_core_info()
    nc = info.num_cores
    nw = nc * info.num_subcores
    n = B // nw
    nch = n // _C
    mesh = plsc.VectorSubcoreMesh(core_axis_name="c", subcore_axis_name="s")

    out_type = [jax.ShapeDtypeStruct((B, d), jnp.float32) for d in _DIMS]
    scratch = ([pltpu.VMEM((nch, _C), jnp.int32) for _ in _DIMS]
               + [pltpu.VMEM((n, d), jnp.float32) for d in _DIMS]
               + [pltpu.SemaphoreType.DMA])

    @functools.partial(
        pl.kernel, mesh=mesh, out_type=out_type, scratch_types=scratch,
        compiler_params=pltpu.CompilerParams(use_tc_tiling_on_sc=False))
    def gather(ids0, ids1, ids2, ids3, t0, t1, t2, t3,
               o0, o1, o2, o3, i0, i1, i2, i3, r0, r1, r2, r3, sem):
        wid = lax.axis_index("s") * nc + lax.axis_index("c")
        idx = (i0, i1, i2, i3)
        ids = (ids0, ids1, ids2, ids3)
        tab = (t0, t1, t2, t3)
        rows = (r0, r1, r2, r3)
        outs = (o0, o1, o2, o3)
        crow = wid * nch
        for k in range(4):
            pltpu.sync_copy(ids[k].at[pl.ds(crow, nch)], idx[k])
        copies = []
        for j in range(nch):
            for k in range(4):
                copies.append(pltpu.async_copy(
                    tab[k].at[idx[k].at[j]],
                    rows[k].at[pl.ds(j * _C, _C)], sem))
        for c in copies:
            c.wait()
        base = wid * n
        for k in range(4):
            pltpu.sync_copy(rows[k], outs[k].at[pl.ds(base, n)])

    return gather


def _mlp_body(item_r, br_r, ct_r, sl_r, ft_r,
              wi_r, wb_r, wc_r, ws_r, wf_r, b1_r, w2_r, b2_r, out_r):
    f32 = jnp.float32
    h = jnp.dot(item_r[...], wi_r[...], preferred_element_type=f32)
    h = h + jnp.dot(br_r[...], wb_r[...], preferred_element_type=f32)
    h = h + jnp.dot(ct_r[...], wc_r[...], preferred_element_type=f32)
    h = h + jnp.dot(sl_r[...], ws_r[...], preferred_element_type=f32)
    h = h + jnp.dot(ft_r[...], wf_r[...], preferred_element_type=f32)
    h = jnp.maximum(h + b1_r[...], 0.0)
    out_r[...] = jnp.dot(h, w2_r[...], preferred_element_type=f32) + b2_r[...]


def kernel(item_ids, item_features, item_table, emb_brand, emb_category,
           emb_seller, W1, b1, W2, b2):
    B = item_ids.shape[0]
    ids = item_ids.astype(jnp.int32).reshape(B // _C, _C)
    f_idx = item_features[:, :3].astype(jnp.int32)
    brand = f_idx[:, 0].reshape(B // _C, _C)
    cat = f_idx[:, 1].reshape(B // _C, _C)
    sell = f_idx[:, 2].reshape(B // _C, _C)
    feats = item_features[:, 3:37]

    item_e, br_e, ct_e, sl_e = _make_gather(B)(
        ids, brand, cat, sell, item_table, emb_brand, emb_category, emb_seller)

    wi = W1[:, 0:64].T
    wb = W1[:, 64:96].T
    wc = W1[:, 96:112].T
    ws = W1[:, 112:144].T
    wf = W1[:, 144:178].T
    b1r = b1.reshape(1, -1)
    b2r = b2.reshape(1, -1)

    bt = 1024
    H = W1.shape[0]
    O = W2.shape[0]
    out = pl.pallas_call(
        _mlp_body,
        grid=(B // bt,),
        in_specs=[
            pl.BlockSpec((bt, 64), lambda i: (i, 0)),
            pl.BlockSpec((bt, 32), lambda i: (i, 0)),
            pl.BlockSpec((bt, 16), lambda i: (i, 0)),
            pl.BlockSpec((bt, 32), lambda i: (i, 0)),
            pl.BlockSpec((bt, 34), lambda i: (i, 0)),
            pl.BlockSpec((64, H), lambda i: (0, 0)),
            pl.BlockSpec((32, H), lambda i: (0, 0)),
            pl.BlockSpec((16, H), lambda i: (0, 0)),
            pl.BlockSpec((32, H), lambda i: (0, 0)),
            pl.BlockSpec((34, H), lambda i: (0, 0)),
            pl.BlockSpec((1, H), lambda i: (0, 0)),
            pl.BlockSpec((H, O), lambda i: (0, 0)),
            pl.BlockSpec((1, O), lambda i: (0, 0)),
        ],
        out_specs=pl.BlockSpec((bt, O), lambda i: (i, 0)),
        out_shape=jax.ShapeDtypeStruct((B, O), jnp.float32),
    )(item_e, br_e, ct_e, sl_e, feats, wi, wb, wc, ws, wf, b1r, W2.T, b2r)
    return out
